# baseline (speedup 1.0000x reference)
import functools

import jax
import jax.numpy as jnp
from jax.experimental import pallas as pl
from jax.experimental.pallas import tpu as pltpu

E = 8
TOP_K = 2
N_GROUP = 4
GROUP_SIZE = E // N_GROUP
HIDDEN = 1024
INTER = 512
T = 2048


def _gating_body(x_ref, rw_ref, bias_ref, comb_ref):
    lg = jax.lax.dot_general(
        rw_ref[...], x_ref[...], (((1,), (1,)), ((), ())),
        preferred_element_type=jnp.float32)
    s = jax.nn.sigmoid(lg)
    rows = [s[e:e + 1, :] for e in range(E)]
    sfc = [rows[e] + bias_ref[e] for e in range(E)]
    g = [sfc[2 * i] + sfc[2 * i + 1] for i in range(N_GROUP)]
    sel = []
    for i in range(N_GROUP):
        r = jnp.zeros_like(g[i])
        for j in range(N_GROUP):
            if j == i:
                continue
            gt = g[j] > g[i]
            if j < i:
                gt = gt | (g[j] == g[i])
            r = r + gt.astype(jnp.float32)
        sel.append(r < float(TOP_K))
    tmp = [jnp.where(sel[e // GROUP_SIZE], sfc[e], 0.0) for e in range(E)]
    cho = []
    for i in range(E):
        r = jnp.zeros_like(tmp[i])
        for j in range(E):
            if j == i:
                continue
            gt = tmp[j] > tmp[i]
            if j < i:
                gt = gt | (tmp[j] == tmp[i])
            r = r + gt.astype(jnp.float32)
        cho.append(r < float(TOP_K))
    w = [jnp.where(cho[e], rows[e], 0.0) for e in range(E)]
    denom = w[0]
    for e in range(1, E):
        denom = denom + w[e]
    denom = denom + 1e-20
    for e in range(E):
        comb_ref[e:e + 1, :] = w[e] / denom


def _ffn_body(comb_ref, x_ref, gw_ref, uw_ref, dw_ref, out_ref, xb_ref):
    e = pl.program_id(0)
    ih = pl.program_id(1)

    @pl.when((e == 0) & (ih == 0))
    def _init():
        out_ref[...] = jnp.zeros_like(out_ref)
        xb_ref[...] = x_ref[...].astype(jnp.bfloat16)

    x = xb_ref[...]
    g = jax.lax.dot_general(x, gw_ref[0].astype(jnp.bfloat16),
                            (((1,), (1,)), ((), ())),
                            preferred_element_type=jnp.float32)
    u = jax.lax.dot_general(x, uw_ref[0].astype(jnp.bfloat16),
                            (((1,), (1,)), ((), ())),
                            preferred_element_type=jnp.float32)
    lane = jax.lax.broadcasted_iota(jnp.int32, (1, E), 1)
    col = jnp.sum(comb_ref[...] * (lane == e).astype(jnp.float32),
                  axis=1, keepdims=True)
    a = (g * jax.nn.sigmoid(g)) * u * col
    y = jax.lax.dot_general(a.astype(jnp.bfloat16),
                            dw_ref[0].astype(jnp.bfloat16),
                            (((1,), (1,)), ((), ())),
                            preferred_element_type=jnp.float32)
    out_ref[...] += y


def _gating(x, router_weight, e_bias):
    return pl.pallas_call(
        _gating_body,
        out_shape=jax.ShapeDtypeStruct((E, T), jnp.float32),
        in_specs=[
            pl.BlockSpec((T, HIDDEN), lambda: (0, 0)),
            pl.BlockSpec((E, HIDDEN), lambda: (0, 0)),
            pl.BlockSpec(memory_space=pltpu.SMEM),
        ],
        out_specs=pl.BlockSpec((E, T), lambda: (0, 0)),
    )(x, router_weight, e_bias)


def _ffn(comb, x, gate_proj, up_proj, down_proj):
    return pl.pallas_call(
        _ffn_body,
        grid=(E, 2),
        out_shape=jax.ShapeDtypeStruct((T, HIDDEN), jnp.float32),
        in_specs=[
            pl.BlockSpec((T, E), lambda e, i: (0, 0)),
            pl.BlockSpec((T, HIDDEN), lambda e, i: (0, 0)),
            pl.BlockSpec((1, INTER // 2, HIDDEN), lambda e, i: (e, i, 0)),
            pl.BlockSpec((1, INTER // 2, HIDDEN), lambda e, i: (e, i, 0)),
            pl.BlockSpec((1, HIDDEN, INTER // 2), lambda e, i: (e, 0, i)),
        ],
        out_specs=pl.BlockSpec((T, HIDDEN), lambda e, i: (0, 0)),
        scratch_shapes=[pltpu.VMEM((T, HIDDEN), jnp.bfloat16)],
    )(comb, x, gate_proj, up_proj, down_proj)


def kernel(hidden_states, router_weight, e_bias, gate_proj, up_proj, down_proj):
    bsz, seq_len, h = hidden_states.shape
    x = hidden_states.reshape(-1, h).astype(jnp.float32)
    comb_t = _gating(x, router_weight, e_bias)
    comb = comb_t.T
    out = _ffn(comb, x, gate_proj, up_proj, down_proj)
    return out.reshape(bsz, seq_len, h)


# speedup vs baseline: 1.0289x; 1.0289x over previous
import functools

import jax
import jax.numpy as jnp
from jax.experimental import pallas as pl
from jax.experimental.pallas import tpu as pltpu

E = 8
TOP_K = 2
N_GROUP = 4
GROUP_SIZE = E // N_GROUP
HIDDEN = 1024
INTER = 512
T = 2048


def _gating_body(x_ref, rw_ref, bias_ref, comb_ref):
    lg = jax.lax.dot_general(
        rw_ref[...], x_ref[...], (((1,), (1,)), ((), ())),
        preferred_element_type=jnp.float32)
    s = jax.nn.sigmoid(lg)
    rows = [s[e:e + 1, :] for e in range(E)]
    sfc = [rows[e] + bias_ref[e] for e in range(E)]
    g = [sfc[2 * i] + sfc[2 * i + 1] for i in range(N_GROUP)]
    sel = []
    for i in range(N_GROUP):
        r = jnp.zeros_like(g[i])
        for j in range(N_GROUP):
            if j == i:
                continue
            gt = g[j] > g[i]
            if j < i:
                gt = gt | (g[j] == g[i])
            r = r + gt.astype(jnp.float32)
        sel.append(r < float(TOP_K))
    tmp = [jnp.where(sel[e // GROUP_SIZE], sfc[e], 0.0) for e in range(E)]
    cho = []
    for i in range(E):
        r = jnp.zeros_like(tmp[i])
        for j in range(E):
            if j == i:
                continue
            gt = tmp[j] > tmp[i]
            if j < i:
                gt = gt | (tmp[j] == tmp[i])
            r = r + gt.astype(jnp.float32)
        cho.append(r < float(TOP_K))
    w = [jnp.where(cho[e], rows[e], 0.0) for e in range(E)]
    denom = w[0]
    for e in range(1, E):
        denom = denom + w[e]
    denom = denom + 1e-20
    for e in range(E):
        comb_ref[e:e + 1, :] = w[e] / denom


def _ffn_body(comb_ref, x_ref, gw_ref, uw_ref, dw_ref, out_ref):
    e = pl.program_id(0)
    ih = pl.program_id(1)

    @pl.when((e == 0) & (ih == 0))
    def _init():
        out_ref[...] = jnp.zeros_like(out_ref)

    x = x_ref[...].astype(jnp.bfloat16)
    g = jax.lax.dot_general(x, gw_ref[0].astype(jnp.bfloat16),
                            (((1,), (1,)), ((), ())),
                            preferred_element_type=jnp.float32)
    u = jax.lax.dot_general(x, uw_ref[0].astype(jnp.bfloat16),
                            (((1,), (1,)), ((), ())),
                            preferred_element_type=jnp.float32)
    a = (g * jax.nn.sigmoid(g)) * u
    y = jax.lax.dot_general(a.astype(jnp.bfloat16),
                            dw_ref[0].astype(jnp.bfloat16),
                            (((1,), (1,)), ((), ())),
                            preferred_element_type=jnp.float32)
    lane = jax.lax.broadcasted_iota(jnp.int32, (1, E), 1)
    col = jnp.sum(comb_ref[...] * (lane == e).astype(jnp.float32),
                  axis=1, keepdims=True)
    out_ref[...] += y * col


def _gating(x, router_weight, e_bias):
    return pl.pallas_call(
        _gating_body,
        out_shape=jax.ShapeDtypeStruct((E, T), jnp.float32),
        in_specs=[
            pl.BlockSpec((T, HIDDEN), lambda: (0, 0)),
            pl.BlockSpec((E, HIDDEN), lambda: (0, 0)),
            pl.BlockSpec(memory_space=pltpu.SMEM),
        ],
        out_specs=pl.BlockSpec((E, T), lambda: (0, 0)),
    )(x, router_weight, e_bias)


def _ffn(comb, x, gate_proj, up_proj, down_proj):
    return pl.pallas_call(
        _ffn_body,
        grid=(E, 2),
        out_shape=jax.ShapeDtypeStruct((T, HIDDEN), jnp.float32),
        in_specs=[
            pl.BlockSpec((T, E), lambda e, i: (0, 0)),
            pl.BlockSpec((T, HIDDEN), lambda e, i: (0, 0)),
            pl.BlockSpec((1, INTER // 2, HIDDEN), lambda e, i: (e, i, 0)),
            pl.BlockSpec((1, INTER // 2, HIDDEN), lambda e, i: (e, i, 0)),
            pl.BlockSpec((1, HIDDEN, INTER // 2), lambda e, i: (e, 0, i)),
        ],
        out_specs=pl.BlockSpec((T, HIDDEN), lambda e, i: (0, 0)),
    )(comb, x, gate_proj, up_proj, down_proj)


def kernel(hidden_states, router_weight, e_bias, gate_proj, up_proj, down_proj):
    bsz, seq_len, h = hidden_states.shape
    x = hidden_states.reshape(-1, h).astype(jnp.float32)
    comb_t = _gating(x, router_weight, e_bias)
    comb = comb_t.T
    out = _ffn(comb, x, gate_proj, up_proj, down_proj)
    return out.reshape(bsz, seq_len, h)
